# baseline (speedup 1.0000x reference)
import functools
import math

import jax
import jax.numpy as jnp
from jax.experimental import pallas as pl
from jax.experimental.pallas import tpu as pltpu

_E = 8
_TOPK = 2
_G = 8


def _gate_kernel(x_ref, wg_ref, bg_ref, prob_ref, gate_ref, cnt_ref, sent_ref):
    x = x_ref[...]
    B, S, _D = x.shape
    Ex = wg_ref.shape[1]
    xa = jnp.sum(x, axis=1) * (1.0 / S)
    logits = jax.lax.dot_general(
        xa, wg_ref[...], (((1,), (0,)), ((), ())),
        preferred_element_type=jnp.float32) + bg_ref[...]
    m = jnp.max(logits, axis=1, keepdims=True)
    ex = jnp.exp(logits - m)
    p = ex / jnp.sum(ex, axis=1, keepdims=True)

    idx = jax.lax.broadcasted_iota(jnp.int32, (B, Ex), 1)
    v1 = jnp.max(p, axis=1, keepdims=True)
    i1 = jnp.min(jnp.where(p == v1, idx, Ex), axis=1, keepdims=True)
    pm = jnp.where(idx == i1, -1.0, p)
    v2 = jnp.max(pm, axis=1, keepdims=True)
    i2 = jnp.min(jnp.where(pm == v2, idx, Ex), axis=1, keepdims=True)
    prob_ref[...] = jnp.concatenate([v1, v2], axis=1)
    gate_ref[...] = jnp.concatenate([i1, i2], axis=1)

    M = jnp.logical_or(idx == i1, idx == i2).astype(jnp.float32)
    Mt = jnp.transpose(M)
    cnt_ref[...] = jnp.sum(Mt, axis=1, keepdims=True).astype(jnp.int32)
    bi = jax.lax.broadcasted_iota(jnp.int32, (B, B), 0)
    bj = jax.lax.broadcasted_iota(jnp.int32, (B, B), 1)
    U = (bi <= bj).astype(jnp.float32)
    pos1 = jax.lax.dot_general(Mt, U, (((1,), (0,)), ((), ())),
                               preferred_element_type=jnp.float32)
    parr = jax.lax.broadcasted_iota(jnp.int32, (Ex, B, B), 1).astype(jnp.float32)
    barr = jax.lax.broadcasted_iota(jnp.int32, (Ex, B, B), 2).astype(jnp.float32)
    oh = jnp.where((pos1[:, None, :] == parr + 1.0) & (Mt[:, None, :] > 0.0),
                   barr, 0.0)
    sent_ref[...] = jnp.sum(oh, axis=2).astype(jnp.int32)


def _ffn_kernel(cnt_ref, sent_ref, x_ref, w1_ref, b1_ref, w2_ref, b2_ref,
                out_ref, xs_ref, *, seq: int, g: int):
    e = pl.program_id(0)

    @pl.when(e == 0)
    def _init():
        out_ref[...] = jnp.zeros_like(out_ref)

    cnt = cnt_ref[e, 0]
    w1 = w1_ref[0]
    w2 = w2_ref[0]
    b1 = b1_ref[0]
    b2 = b2_ref[0]
    nch = jax.lax.div(cnt + (g - 1), g)

    def gather(c, buf):
        base = c * g
        for k in range(g):
            pidx = base + k
            pc = jnp.minimum(pidx, cnt - 1)
            s = sent_ref[e, pc]
            xs_ref[buf, pl.ds(k * seq, seq), :] = (
                x_ref[pl.ds(s * seq, seq), :])

    @pl.when(nch > 0)
    def _prologue():
        gather(0, 0)

    def body(c, carry):
        @pl.when(c + 1 < nch)
        def _stage():
            gather(c + 1, (c + 1) & 1)

        X = xs_ref[c & 1]
        h = jax.lax.dot_general(X, w1, (((1,), (0,)), ((), ())),
                                preferred_element_type=jnp.float32) + b1
        h = 0.5 * h * (1.0 + jax.lax.erf(h * (1.0 / math.sqrt(2.0))))
        y = jax.lax.dot_general(h, w2, (((1,), (0,)), ((), ())),
                                preferred_element_type=jnp.float32) + b2
        base = c * g
        for k in range(g):
            pidx = base + k
            pc = jnp.minimum(pidx, cnt - 1)
            s = sent_ref[e, pc]
            yk = y[k * seq:(k + 1) * seq, :]

            @pl.when(pidx < cnt)
            def _store(s=s, yk=yk):
                out_ref[pl.ds(s * seq, seq), :] += yk
        return carry

    jax.lax.fori_loop(0, nch, body, 0)


def kernel(x, attention_mask, Wg, bg, W1, b1, W2, b2):
    B, S, D = x.shape
    Ex = Wg.shape[1]

    prob, gate, cnt, sent = pl.pallas_call(
        _gate_kernel,
        out_shape=(
            jax.ShapeDtypeStruct((B, _TOPK), jnp.float32),
            jax.ShapeDtypeStruct((B, _TOPK), jnp.int32),
            jax.ShapeDtypeStruct((Ex, 1), jnp.int32),
            jax.ShapeDtypeStruct((Ex, B), jnp.int32),
        ),
    )(x, Wg, bg.reshape(1, Ex))

    x2 = x.reshape(B * S, D)
    out = pl.pallas_call(
        functools.partial(_ffn_kernel, seq=S, g=_G),
        grid=(Ex,),
        in_specs=[
            pl.BlockSpec(memory_space=pltpu.SMEM),
            pl.BlockSpec(memory_space=pltpu.SMEM),
            pl.BlockSpec((B * S, D), lambda e: (0, 0)),
            pl.BlockSpec((1, D, D), lambda e: (e, 0, 0)),
            pl.BlockSpec((1, 1, D), lambda e: (e, 0, 0)),
            pl.BlockSpec((1, D, D), lambda e: (e, 0, 0)),
            pl.BlockSpec((1, 1, D), lambda e: (e, 0, 0)),
        ],
        out_specs=pl.BlockSpec((B * S, D), lambda e: (0, 0)),
        out_shape=jax.ShapeDtypeStruct((B * S, D), jnp.float32),
        scratch_shapes=[pltpu.VMEM((2, _G * S, D), jnp.float32)],
        compiler_params=pltpu.CompilerParams(
            dimension_semantics=("arbitrary",)),
    )(cnt, sent, x2, W1, b1.reshape(Ex, 1, D), W2, b2.reshape(Ex, 1, D))

    return out.reshape(B, S, D), prob, gate


# speedup vs baseline: 1.0175x; 1.0175x over previous
import functools
import math

import jax
import jax.numpy as jnp
from jax.experimental import pallas as pl
from jax.experimental.pallas import tpu as pltpu

_E = 8
_TOPK = 2
_G = 8


def _gate_kernel(x_ref, wg_ref, bg_ref, prob_ref, gate_ref, cnt_ref, sent_ref,
                 xa_ref, *, seq: int, nsb: int):
    sb = pl.program_id(0)
    xblk = x_ref[...]
    part = jnp.sum(xblk, axis=1)

    @pl.when(sb == 0)
    def _first():
        xa_ref[...] = part

    @pl.when(sb > 0)
    def _acc():
        xa_ref[...] += part

    @pl.when(sb == nsb - 1)
    def _finish():
        _gate_finish(wg_ref, bg_ref, prob_ref, gate_ref, cnt_ref, sent_ref,
                     xa_ref, seq)


def _gate_finish(wg_ref, bg_ref, prob_ref, gate_ref, cnt_ref, sent_ref,
                 xa_ref, seq):
    B = xa_ref.shape[0]
    Ex = wg_ref.shape[1]
    xa = xa_ref[...] * (1.0 / seq)
    logits = jax.lax.dot_general(
        xa, wg_ref[...], (((1,), (0,)), ((), ())),
        preferred_element_type=jnp.float32) + bg_ref[...]
    m = jnp.max(logits, axis=1, keepdims=True)
    ex = jnp.exp(logits - m)
    p = ex / jnp.sum(ex, axis=1, keepdims=True)

    idx = jax.lax.broadcasted_iota(jnp.int32, (B, Ex), 1)
    v1 = jnp.max(p, axis=1, keepdims=True)
    i1 = jnp.min(jnp.where(p == v1, idx, Ex), axis=1, keepdims=True)
    pm = jnp.where(idx == i1, -1.0, p)
    v2 = jnp.max(pm, axis=1, keepdims=True)
    i2 = jnp.min(jnp.where(pm == v2, idx, Ex), axis=1, keepdims=True)
    prob_ref[...] = jnp.concatenate([v1, v2], axis=1)
    gate_ref[...] = jnp.concatenate([i1, i2], axis=1)

    M = jnp.logical_or(idx == i1, idx == i2).astype(jnp.float32)
    Mt = jnp.transpose(M)
    cnt_ref[...] = jnp.sum(Mt, axis=1, keepdims=True).astype(jnp.int32)
    bi = jax.lax.broadcasted_iota(jnp.int32, (B, B), 0)
    bj = jax.lax.broadcasted_iota(jnp.int32, (B, B), 1)
    U = (bi <= bj).astype(jnp.float32)
    pos1 = jax.lax.dot_general(Mt, U, (((1,), (0,)), ((), ())),
                               preferred_element_type=jnp.float32)
    parr = jax.lax.broadcasted_iota(jnp.int32, (Ex, B, B), 1).astype(jnp.float32)
    barr = jax.lax.broadcasted_iota(jnp.int32, (Ex, B, B), 2).astype(jnp.float32)
    oh = jnp.where((pos1[:, None, :] == parr + 1.0) & (Mt[:, None, :] > 0.0),
                   barr, 0.0)
    sent_ref[...] = jnp.sum(oh, axis=2).astype(jnp.int32)


def _ffn_kernel(cnt_ref, sent_ref, x_ref, w1_ref, b1_ref, w2_ref, b2_ref,
                out_ref, xs_ref, *, seq: int, g: int):
    e = pl.program_id(0)

    @pl.when(e == 0)
    def _init():
        out_ref[...] = jnp.zeros_like(out_ref)

    cnt = cnt_ref[e, 0]
    w1 = w1_ref[0]
    w2 = w2_ref[0]
    b1 = b1_ref[0]
    b2 = b2_ref[0]
    nch = jax.lax.div(cnt + (g - 1), g)

    def body(c, carry):
        base = c * g
        picks = []
        for k in range(g):
            pidx = base + k
            pc = jnp.minimum(pidx, cnt - 1)
            s = sent_ref[e, pc]
            picks.append((s, pidx < cnt))
            xs_ref[pl.ds(k * seq, seq), :] = (
                x_ref[pl.ds(s * seq, seq), :])
        X = xs_ref[...]
        h = jax.lax.dot_general(X, w1, (((1,), (0,)), ((), ())),
                                preferred_element_type=jnp.float32) + b1
        h = 0.5 * h * (1.0 + jax.lax.erf(h * (1.0 / math.sqrt(2.0))))
        y = jax.lax.dot_general(h, w2, (((1,), (0,)), ((), ())),
                                preferred_element_type=jnp.float32) + b2
        for k in range(g):
            s, valid = picks[k]
            yk = y[k * seq:(k + 1) * seq, :]

            @pl.when(valid)
            def _store(s=s, yk=yk):
                out_ref[pl.ds(s * seq, seq), :] += yk
        return carry

    jax.lax.fori_loop(0, nch, body, 0)


def kernel(x, attention_mask, Wg, bg, W1, b1, W2, b2):
    B, S, D = x.shape
    Ex = Wg.shape[1]

    nsb = 4
    sc = S // nsb
    prob, gate, cnt, sent = pl.pallas_call(
        functools.partial(_gate_kernel, seq=S, nsb=nsb),
        grid=(nsb,),
        in_specs=[
            pl.BlockSpec((B, sc, D), lambda s: (0, s, 0)),
            pl.BlockSpec((D, Ex), lambda s: (0, 0)),
            pl.BlockSpec((1, Ex), lambda s: (0, 0)),
        ],
        out_specs=(
            pl.BlockSpec((B, _TOPK), lambda s: (0, 0)),
            pl.BlockSpec((B, _TOPK), lambda s: (0, 0)),
            pl.BlockSpec((Ex, 1), lambda s: (0, 0)),
            pl.BlockSpec((Ex, B), lambda s: (0, 0)),
        ),
        out_shape=(
            jax.ShapeDtypeStruct((B, _TOPK), jnp.float32),
            jax.ShapeDtypeStruct((B, _TOPK), jnp.int32),
            jax.ShapeDtypeStruct((Ex, 1), jnp.int32),
            jax.ShapeDtypeStruct((Ex, B), jnp.int32),
        ),
        scratch_shapes=[pltpu.VMEM((B, D), jnp.float32)],
        compiler_params=pltpu.CompilerParams(
            dimension_semantics=("arbitrary",)),
    )(x, Wg, bg.reshape(1, Ex))

    x2 = x.reshape(B * S, D)
    out = pl.pallas_call(
        functools.partial(_ffn_kernel, seq=S, g=_G),
        grid=(Ex,),
        in_specs=[
            pl.BlockSpec(memory_space=pltpu.SMEM),
            pl.BlockSpec(memory_space=pltpu.SMEM),
            pl.BlockSpec((B * S, D), lambda e: (0, 0)),
            pl.BlockSpec((1, D, D), lambda e: (e, 0, 0)),
            pl.BlockSpec((1, 1, D), lambda e: (e, 0, 0)),
            pl.BlockSpec((1, D, D), lambda e: (e, 0, 0)),
            pl.BlockSpec((1, 1, D), lambda e: (e, 0, 0)),
        ],
        out_specs=pl.BlockSpec((B * S, D), lambda e: (0, 0)),
        out_shape=jax.ShapeDtypeStruct((B * S, D), jnp.float32),
        scratch_shapes=[pltpu.VMEM((_G * S, D), jnp.float32)],
        compiler_params=pltpu.CompilerParams(
            dimension_semantics=("arbitrary",)),
    )(cnt, sent, x2, W1, b1.reshape(Ex, 1, D), W2, b2.reshape(Ex, 1, D))

    return out.reshape(B, S, D), prob, gate


# speedup vs baseline: 1.0653x; 1.0469x over previous
import functools
import math

import jax
import jax.numpy as jnp
from jax.experimental import pallas as pl
from jax.experimental.pallas import tpu as pltpu

_E = 8
_TOPK = 2
_G = 8


def _gate_kernel(x_ref, wg_ref, bg_ref, prob_ref, gate_ref, cnt_ref, sent_ref):
    x = x_ref[...]
    B, S, _D = x.shape
    Ex = wg_ref.shape[1]
    xa = jnp.sum(x, axis=1) * (1.0 / S)
    logits = jax.lax.dot_general(
        xa, wg_ref[...], (((1,), (0,)), ((), ())),
        preferred_element_type=jnp.float32) + bg_ref[...]
    m = jnp.max(logits, axis=1, keepdims=True)
    ex = jnp.exp(logits - m)
    p = ex / jnp.sum(ex, axis=1, keepdims=True)

    idx = jax.lax.broadcasted_iota(jnp.int32, (B, Ex), 1)
    v1 = jnp.max(p, axis=1, keepdims=True)
    i1 = jnp.min(jnp.where(p == v1, idx, Ex), axis=1, keepdims=True)
    pm = jnp.where(idx == i1, -1.0, p)
    v2 = jnp.max(pm, axis=1, keepdims=True)
    i2 = jnp.min(jnp.where(pm == v2, idx, Ex), axis=1, keepdims=True)
    prob_ref[...] = jnp.concatenate([v1, v2], axis=1)
    gate_ref[...] = jnp.concatenate([i1, i2], axis=1)

    M = jnp.logical_or(idx == i1, idx == i2).astype(jnp.float32)
    Mt = jnp.transpose(M)
    cnt_ref[...] = jnp.sum(Mt, axis=1, keepdims=True).astype(jnp.int32)
    bi = jax.lax.broadcasted_iota(jnp.int32, (B, B), 0)
    bj = jax.lax.broadcasted_iota(jnp.int32, (B, B), 1)
    U = (bi <= bj).astype(jnp.float32)
    pos1 = jax.lax.dot_general(Mt, U, (((1,), (0,)), ((), ())),
                               preferred_element_type=jnp.float32)
    parr = jax.lax.broadcasted_iota(jnp.int32, (Ex, B, B), 1).astype(jnp.float32)
    barr = jax.lax.broadcasted_iota(jnp.int32, (Ex, B, B), 2).astype(jnp.float32)
    oh = jnp.where((pos1[:, None, :] == parr + 1.0) & (Mt[:, None, :] > 0.0),
                   barr, 0.0)
    sent_ref[...] = jnp.sum(oh, axis=2).astype(jnp.int32)


def _ffn_kernel(cnt_ref, sent_ref, x_ref, w1_ref, b1_ref, w2_ref, b2_ref,
                out_ref, xs_ref, *, seq: int, g: int):
    e = pl.program_id(0)

    @pl.when(e == 0)
    def _init():
        out_ref[...] = jnp.zeros_like(out_ref)

    cnt = cnt_ref[e, 0]
    w1 = w1_ref[0]
    w2 = w2_ref[0]
    b1 = b1_ref[0]
    b2 = b2_ref[0]
    nch = jax.lax.div(cnt + (g - 1), g)
    nfull = jax.lax.div(cnt, g)
    npair = jax.lax.div(nfull, 2)

    def ffn(X):
        h = jax.lax.dot_general(X, w1, (((1,), (0,)), ((), ())),
                                preferred_element_type=jnp.float32) + b1
        h = 0.5 * h * (1.0 + jax.lax.erf(h * (1.0 / math.sqrt(2.0))))
        return jax.lax.dot_general(h, w2, (((1,), (0,)), ((), ())),
                                   preferred_element_type=jnp.float32) + b2

    def load_chunk(c, buf):
        base = c * g
        for k in range(g):
            s = sent_ref[e, base + k]
            xs_ref[buf, pl.ds(k * seq, seq), :] = (
                x_ref[pl.ds(s * seq, seq), :])

    def scatter(c, y):
        base = c * g
        for k in range(g):
            s = sent_ref[e, base + k]
            out_ref[pl.ds(s * seq, seq), :] += y[k * seq:(k + 1) * seq, :]

    def pair_body(i, carry):
        cA = 2 * i
        load_chunk(cA, 0)
        load_chunk(cA + 1, 1)
        scatter(cA, ffn(xs_ref[0]))
        scatter(cA + 1, ffn(xs_ref[1]))
        return carry

    jax.lax.fori_loop(0, npair, pair_body, 0)

    def tail_body(c, carry):
        base = c * g
        picks = []
        for k in range(g):
            pidx = base + k
            pc = jnp.minimum(pidx, cnt - 1)
            s = sent_ref[e, pc]
            picks.append((s, pidx < cnt))
            xs_ref[0, pl.ds(k * seq, seq), :] = (
                x_ref[pl.ds(s * seq, seq), :])
        y = ffn(xs_ref[0])
        for k in range(g):
            s, valid = picks[k]
            yk = y[k * seq:(k + 1) * seq, :]

            @pl.when(valid)
            def _store(s=s, yk=yk):
                out_ref[pl.ds(s * seq, seq), :] += yk
        return carry

    jax.lax.fori_loop(2 * npair, nch, tail_body, 0)


def kernel(x, attention_mask, Wg, bg, W1, b1, W2, b2):
    B, S, D = x.shape
    Ex = Wg.shape[1]

    prob, gate, cnt, sent = pl.pallas_call(
        _gate_kernel,
        out_shape=(
            jax.ShapeDtypeStruct((B, _TOPK), jnp.float32),
            jax.ShapeDtypeStruct((B, _TOPK), jnp.int32),
            jax.ShapeDtypeStruct((Ex, 1), jnp.int32),
            jax.ShapeDtypeStruct((Ex, B), jnp.int32),
        ),
    )(x, Wg, bg.reshape(1, Ex))

    x2 = x.reshape(B * S, D)
    out = pl.pallas_call(
        functools.partial(_ffn_kernel, seq=S, g=_G),
        grid=(Ex,),
        in_specs=[
            pl.BlockSpec(memory_space=pltpu.SMEM),
            pl.BlockSpec(memory_space=pltpu.SMEM),
            pl.BlockSpec((B * S, D), lambda e: (0, 0)),
            pl.BlockSpec((1, D, D), lambda e: (e, 0, 0)),
            pl.BlockSpec((1, 1, D), lambda e: (e, 0, 0)),
            pl.BlockSpec((1, D, D), lambda e: (e, 0, 0)),
            pl.BlockSpec((1, 1, D), lambda e: (e, 0, 0)),
        ],
        out_specs=pl.BlockSpec((B * S, D), lambda e: (0, 0)),
        out_shape=jax.ShapeDtypeStruct((B * S, D), jnp.float32),
        scratch_shapes=[pltpu.VMEM((2, _G * S, D), jnp.float32)],
        compiler_params=pltpu.CompilerParams(
            dimension_semantics=("arbitrary",)),
    )(cnt, sent, x2, W1, b1.reshape(Ex, 1, D), W2, b2.reshape(Ex, 1, D))

    return out.reshape(B, S, D), prob, gate
